# Initial kernel scaffold; baseline (speedup 1.0000x reference)
import jax
import jax.numpy as jnp
from jax.experimental import pallas as pl


def kernel(x_1, edge_index_1, x_2, edge_index_2, params1, params2, lin):
    raise NotImplementedError("write your pallas kernel here")


# speedup vs baseline: 50.8053x; 50.8053x over previous
import functools

import jax
import jax.numpy as jnp
from jax import lax
from jax.experimental import pallas as pl
from jax.experimental.pallas import tpu as pltpu
from jax.experimental.pallas import tpu_sc as plsc

_N = 50000
_IN = 16
_H = 32
_T = 4
_E = 800000

_NPAD = 51200
_R = 3200
_EPAD = 802816
_RIDX = _EPAD // 128
_ROWS_P1 = _RIDX // 16
_ROWS_P2 = _RIDX // 32

_f32 = jnp.float32


def _mesh():
    return plsc.VectorSubcoreMesh(
        core_axis_name="c", subcore_axis_name="s", num_cores=2, num_subcores=16)


def _sc_degree(dst_r, z1):
    @functools.partial(
        pl.kernel,
        out_type=jax.ShapeDtypeStruct((2, 1, _NPAD), _f32),
        mesh=_mesh(),
        scratch_types=[
            pltpu.VMEM((1, 128), jnp.int32),
            pltpu.VMEM((128,), _f32),
            pltpu.VMEM_SHARED((_NPAD,), _f32),
        ],
    )
    def k(dst_hbm, z_hbm, out_hbm, didx, ones_v, acc):
        c = lax.axis_index("c")
        s = lax.axis_index("s")
        for kk in range(8):
            ones_v[kk * 16:(kk + 1) * 16] = jnp.ones((16,), _f32)
        nrows = _NPAD // 16
        pltpu.sync_copy(z_hbm.at[pl.ds(s * nrows, nrows)],
                        acc.at[pl.ds(s * nrows, nrows)])
        plsc.subcore_barrier()
        base = (s * 2 + c) * _ROWS_P2

        def body(j, carry):
            pltpu.sync_copy(dst_hbm.at[base + j], didx.at[0])
            pltpu.sync_copy(ones_v, acc.at[didx.at[0]], add=True)
            return carry

        lax.fori_loop(0, _ROWS_P2, body, 0)
        plsc.subcore_barrier()
        pltpu.sync_copy(acc.at[pl.ds(s * nrows, nrows)],
                        out_hbm.at[c, 0, pl.ds(s * nrows, nrows)])

    return k(dst_r, z1)


def _make_spmv(rows_per_task, by_wid, offset_cores, table_rows):
    @functools.partial(
        pl.kernel,
        out_type=jax.ShapeDtypeStruct((2, _NPAD, 32), _f32),
        mesh=_mesh(),
        scratch_types=[
            pltpu.VMEM((1, 128), jnp.int32),
            pltpu.VMEM((1, 128), jnp.int32),
            pltpu.VMEM((128, 32), _f32),
            pltpu.VMEM_SHARED((_NPAD, 32), _f32),
            pltpu.SemaphoreType.DMA,
        ],
        compiler_params=pltpu.CompilerParams(use_tc_tiling_on_sc=False),
    )
    def k(tab_hbm, src_hbm, dst_hbm, z_hbm, out_hbm, sidx, didx, rows_v, acc, sem):
        c = lax.axis_index("c")
        s = lax.axis_index("s")
        task = (s * 2 + c) if by_wid else s
        nrows = _NPAD // 16
        pltpu.sync_copy(z_hbm.at[pl.ds(s * nrows, nrows)],
                        acc.at[pl.ds(s * nrows, nrows)])
        plsc.subcore_barrier()
        base = task * rows_per_task
        off = c * table_rows

        def body(j, carry):
            pltpu.sync_copy(src_hbm.at[base + j], sidx.at[0])
            pltpu.sync_copy(dst_hbm.at[base + j], didx.at[0])
            if offset_cores:
                for kk in range(8):
                    sidx[0, kk * 16:(kk + 1) * 16] = (
                        sidx[0, kk * 16:(kk + 1) * 16] + off)
            pltpu.async_copy(tab_hbm.at[sidx.at[0]], rows_v, sem).wait()
            pltpu.sync_copy(rows_v, acc.at[didx.at[0]], add=True)
            return carry

        lax.fori_loop(0, rows_per_task, body, 0)
        plsc.subcore_barrier()
        pltpu.sync_copy(acc.at[pl.ds(s * nrows, nrows)],
                        out_hbm.at[c, pl.ds(s * nrows, nrows)])

    return k


def _tc_prescale(xflat, degt):
    def body(x_ref, d_ref, xa_ref, xb_ref):
        d = d_ref[:, 0:1] + d_ref[:, 1:2] + 1.0
        dinv = lax.rsqrt(d)
        x = x_ref[...]
        xa_ref[...] = x[:, 0:32] * dinv
        xb_ref[...] = x[:, 32:64] * dinv

    return pl.pallas_call(
        body,
        grid=(_NPAD // _R,),
        in_specs=[
            pl.BlockSpec((_R, 64), lambda i: (i, 0)),
            pl.BlockSpec((_R, 2), lambda i: (i, 0)),
        ],
        out_specs=[pl.BlockSpec((_R, 32), lambda i: (i, 0))] * 2,
        out_shape=[jax.ShapeDtypeStruct((_NPAD, 32), _f32)] * 2,
    )(xflat, degt)


def _softmax4(att):
    a = att
    e = jnp.exp(a - jnp.max(a))
    return e / jnp.sum(e)


def _gru_step(g, Hs, lwz, lwr, lwh, lbz, lbr, lbh):
    gz, gr, gh = g[:, 0:32], g[:, 32:64], g[:, 64:96]
    z = jax.nn.sigmoid(
        jnp.dot(jnp.concatenate([gz, Hs], axis=1), lwz,
                preferred_element_type=_f32, precision=lax.Precision.HIGHEST) + lbz)
    r = jax.nn.sigmoid(
        jnp.dot(jnp.concatenate([gr, Hs], axis=1), lwr,
                preferred_element_type=_f32, precision=lax.Precision.HIGHEST) + lbr)
    ht = jnp.tanh(
        jnp.dot(jnp.concatenate([gh, Hs * r], axis=1), lwh,
                preferred_element_type=_f32, precision=lax.Precision.HIGHEST) + lbh)
    return z * Hs + (1.0 - z) * ht


def _tc_layer1(y1p, xa, xb, degt, wcat, bcat, lwz, lwr, lwh, lbz, lbr, lbh, att):
    def body(y_ref, xa_ref, xb_ref, d_ref, wc_ref, bc_ref, lwz_ref, lwr_ref,
             lwh_ref, lbz_ref, lbr_ref, lbh_ref, att_ref, hq_ref, yt_ref):
        d = d_ref[:, 0:1] + d_ref[:, 1:2] + 1.0
        dinv = lax.rsqrt(d)
        ya = (y_ref[0] + xa_ref[...]) * dinv
        yb = (y_ref[1] + xb_ref[...]) * dinv
        p = _softmax4(att_ref[...])
        wc = wc_ref[...]
        bc = bc_ref[...]
        lwz_, lwr_, lwh_ = lwz_ref[...], lwr_ref[...], lwh_ref[...]
        lbz_, lbr_, lbh_ = lbz_ref[...], lbr_ref[...], lbh_ref[...]
        Hs = jnp.zeros((_R, 32), _f32)
        acc = jnp.zeros((_R, 32), _f32)
        xts = (ya[:, 0:16], ya[:, 16:32], yb[:, 0:16], yb[:, 16:32])
        for t in range(_T):
            g = jnp.dot(xts[t], wc, preferred_element_type=_f32, precision=lax.Precision.HIGHEST) + bc
            Hs = _gru_step(g, Hs, lwz_, lwr_, lwh_, lbz_, lbr_, lbh_)
            acc = acc + p[0:1, t:t + 1] * Hs
        h = jnp.maximum(acc, 0.0)
        hq_ref[...] = h * dinv
        yt_ref[...] = jnp.concatenate(
            [ya[:, 15:16], ya[:, 31:32], yb[:, 15:16], yb[:, 31:32],
             jnp.zeros((_R, 4), _f32)], axis=1)

    full = lambda shape: pl.BlockSpec(shape, lambda i: tuple(0 for _ in shape))
    return pl.pallas_call(
        body,
        grid=(_NPAD // _R,),
        in_specs=[
            pl.BlockSpec((2, _R, 32), lambda i: (0, i, 0)),
            pl.BlockSpec((_R, 32), lambda i: (i, 0)),
            pl.BlockSpec((_R, 32), lambda i: (i, 0)),
            pl.BlockSpec((_R, 2), lambda i: (i, 0)),
            full((16, 96)), full((1, 96)),
            full((64, 32)), full((64, 32)), full((64, 32)),
            full((1, 32)), full((1, 32)), full((1, 32)),
            full((1, 4)),
        ],
        out_specs=[
            pl.BlockSpec((_R, 32), lambda i: (i, 0)),
            pl.BlockSpec((_R, 8), lambda i: (i, 0)),
        ],
        out_shape=[
            jax.ShapeDtypeStruct((_NPAD, 32), _f32),
            jax.ShapeDtypeStruct((_NPAD, 8), _f32),
        ],
    )(y1p, xa, xb, degt, wcat, bcat, lwz, lwr, lwh, lbz, lbr, lbh, att)


def _tc_layer2(y2p, hq, ytime, degt, w2top, w2last, bcat2, lwz, lwr, lwh,
               lbz, lbr, lbh, att2, w1, b1, w2p, b2p):
    def body(y_ref, hq_ref, yt_ref, d_ref, wt_ref, wl_ref, bc_ref,
             lwz_ref, lwr_ref, lwh_ref, lbz_ref, lbr_ref, lbh_ref, att_ref,
             w1_ref, b1_ref, w2_ref, b2_ref, out_ref):
        d = d_ref[:, 0:1] + d_ref[:, 1:2] + 1.0
        dinv = lax.rsqrt(d)
        yh = (y_ref[0] + y_ref[1] + hq_ref[...]) * dinv
        p = _softmax4(att_ref[...])
        gbase = jnp.dot(yh, wt_ref[...], preferred_element_type=_f32, precision=lax.Precision.HIGHEST)
        wl = wl_ref[...]
        bc = bc_ref[...]
        lwz_, lwr_, lwh_ = lwz_ref[...], lwr_ref[...], lwh_ref[...]
        lbz_, lbr_, lbh_ = lbz_ref[...], lbr_ref[...], lbh_ref[...]
        yt = yt_ref[...]
        Hs = jnp.zeros((_R, 32), _f32)
        acc = jnp.zeros((_R, 32), _f32)
        for t in range(_T):
            g = gbase + yt[:, t:t + 1] * wl + bc
            Hs = _gru_step(g, Hs, lwz_, lwr_, lwh_, lbz_, lbr_, lbh_)
            acc = acc + p[0:1, t:t + 1] * Hs
        h = jnp.maximum(acc, 0.0)
        m = jnp.maximum(
            jnp.dot(h, w1_ref[...], preferred_element_type=_f32, precision=lax.Precision.HIGHEST) + b1_ref[...],
            0.0)
        out_ref[...] = (jnp.dot(m, w2_ref[...], preferred_element_type=_f32, precision=lax.Precision.HIGHEST)
                        + b2_ref[...])

    full = lambda shape: pl.BlockSpec(shape, lambda i: tuple(0 for _ in shape))
    return pl.pallas_call(
        body,
        grid=(_NPAD // _R,),
        in_specs=[
            pl.BlockSpec((2, _R, 32), lambda i: (0, i, 0)),
            pl.BlockSpec((_R, 32), lambda i: (i, 0)),
            pl.BlockSpec((_R, 8), lambda i: (i, 0)),
            pl.BlockSpec((_R, 2), lambda i: (i, 0)),
            full((32, 96)), full((1, 96)), full((1, 96)),
            full((64, 32)), full((64, 32)), full((64, 32)),
            full((1, 32)), full((1, 32)), full((1, 32)),
            full((1, 4)),
            full((32, 32)), full((1, 32)), full((32, 8)), full((1, 8)),
        ],
        out_specs=pl.BlockSpec((_R, 8), lambda i: (i, 0)),
        out_shape=jax.ShapeDtypeStruct((_NPAD, 8), _f32),
    )(y2p, hq, ytime, degt, w2top, w2last, bcat2, lwz, lwr, lwh,
      lbz, lbr, lbh, att2, w1, b1, w2p, b2p)


@jax.jit
def _run(x_1, edge_index_1, params1, params2, lin):
    src = edge_index_1[0]
    dst = edge_index_1[1]
    padi = jnp.full((_EPAD - _E,), _N, jnp.int32)
    src_r = jnp.concatenate([src.astype(jnp.int32), padi]).reshape(_RIDX, 128)
    dst_r = jnp.concatenate([dst.astype(jnp.int32), padi]).reshape(_RIDX, 128)

    xflat = x_1.transpose(0, 2, 1).reshape(_N, _T * _IN)
    xflat = jnp.pad(xflat, ((0, _NPAD - _N), (0, 0)))

    z1 = jnp.zeros((_NPAD,), _f32)
    z32 = jnp.zeros((_NPAD, 32), _f32)

    p1, p2 = params1, params2
    wcat1 = jnp.concatenate([p1["Wz"], p1["Wr"], p1["Wh"]], axis=1)
    bcat1 = jnp.concatenate([p1["bz"], p1["br"], p1["bh"]])[None, :]
    att1 = p1["att"][None, :]
    wcat2 = jnp.concatenate([p2["Wz"], p2["Wr"], p2["Wh"]], axis=1)
    w2top = wcat2[0:32]
    w2last = wcat2[32:33]
    bcat2 = jnp.concatenate([p2["bz"], p2["br"], p2["bh"]])[None, :]
    att2 = p2["att"][None, :]
    w2p = jnp.pad(lin["W2"], ((0, 0), (0, 4)))
    b2p = jnp.pad(lin["b2"], (0, 4))[None, :]

    degp = _sc_degree(dst_r, z1)
    degt = degp.reshape(2, _NPAD).T
    xa, xb = _tc_prescale(xflat, degt)
    xab = jnp.concatenate([xa, xb], axis=0)
    y1p = _make_spmv(_ROWS_P1, False, True, _NPAD)(xab, src_r, dst_r, z32)
    hq, ytime = _tc_layer1(y1p, xa, xb, degt, wcat1, bcat1,
                           p1["LWz"], p1["LWr"], p1["LWh"],
                           p1["Lbz"][None, :], p1["Lbr"][None, :],
                           p1["Lbh"][None, :], att1)
    y2p = _make_spmv(_ROWS_P2, True, False, _NPAD)(hq, src_r, dst_r, z32)
    pred = _tc_layer2(y2p, hq, ytime, degt, w2top, w2last, bcat2,
                      p2["LWz"], p2["LWr"], p2["LWh"],
                      p2["Lbz"][None, :], p2["Lbr"][None, :],
                      p2["Lbh"][None, :], att2,
                      lin["W1"], lin["b1"][None, :], w2p, b2p)
    return pred[:_N, 0:4]


def kernel(x_1, edge_index_1, x_2, edge_index_2, params1, params2, lin):
    return _run(x_1, edge_index_1, params1, params2, lin)


# speedup vs baseline: 116.4128x; 2.2914x over previous
import functools

import jax
import jax.numpy as jnp
from jax import lax
from jax.experimental import pallas as pl
from jax.experimental.pallas import tpu as pltpu
from jax.experimental.pallas import tpu_sc as plsc

_N = 50000
_IN = 16
_H = 32
_T = 4
_E = 800000

_NPAD = 51200
_R = 3200
_EPAD = 811008
_RIDX = _EPAD // 128
_ROWS_P1 = _RIDX // 16
_ROWS_P2 = _RIDX // 32
_GRP = 3

_f32 = jnp.float32


def _mesh():
    return plsc.VectorSubcoreMesh(
        core_axis_name="c", subcore_axis_name="s", num_cores=2, num_subcores=16)


def _sc_degree(dst_r, z1):
    @functools.partial(
        pl.kernel,
        out_type=jax.ShapeDtypeStruct((2, 1, _NPAD), _f32),
        mesh=_mesh(),
        scratch_types=[
            pltpu.VMEM((2, _GRP, 128), jnp.int32),
            pltpu.VMEM((128,), _f32),
            pltpu.VMEM_SHARED((_NPAD,), _f32),
            pltpu.SemaphoreType.DMA,
            pltpu.SemaphoreType.DMA,
        ],
        compiler_params=pltpu.CompilerParams(use_tc_tiling_on_sc=False),
    )
    def k(dst_hbm, z_hbm, out_hbm, didx, ones_v, acc, ssem, isem):
        c = lax.axis_index("c")
        s = lax.axis_index("s")
        for kk in range(8):
            ones_v[kk * 16:(kk + 1) * 16] = jnp.ones((16,), _f32)
        nrows = _NPAD // 16
        pltpu.sync_copy(z_hbm.at[pl.ds(s * nrows, nrows)],
                        acc.at[pl.ds(s * nrows, nrows)])
        plsc.subcore_barrier()
        base = (s * 2 + c) * _ROWS_P2
        ngrp = _ROWS_P2 // _GRP
        pltpu.sync_copy(dst_hbm.at[pl.ds(base, _GRP)], didx.at[0])

        def body(g, carry):
            p = g % 2
            pn = (g + 1) % 2

            @pl.when(g >= 1)
            def _():
                pltpu.make_async_copy(
                    dst_hbm.at[pl.ds(base, _GRP)], didx.at[p], isem).wait()
                for u in range(_GRP):
                    pltpu.make_async_copy(
                        z_hbm.at[pl.ds(0, 128)], ones_v, ssem).wait()

            @pl.when(g + 1 < ngrp)
            def _():
                pltpu.async_copy(
                    dst_hbm.at[pl.ds(base + (g + 1) * _GRP, _GRP)],
                    didx.at[pn], isem)
            for u in range(_GRP):
                pltpu.async_copy(ones_v, acc.at[didx.at[p, u]], ssem, add=True)
            return carry

        lax.fori_loop(0, ngrp, body, 0)
        for u in range(_GRP):
            pltpu.make_async_copy(
                z_hbm.at[pl.ds(0, 128)], ones_v, ssem).wait()
        plsc.subcore_barrier()
        pltpu.sync_copy(acc.at[pl.ds(s * nrows, nrows)],
                        out_hbm.at[c, 0, pl.ds(s * nrows, nrows)])

    return k(dst_r, z1)


def _make_spmv(rows_per_task, by_wid):
    @functools.partial(
        pl.kernel,
        out_type=jax.ShapeDtypeStruct((2, _NPAD, 32), _f32),
        mesh=_mesh(),
        scratch_types=[
            pltpu.VMEM((2, _GRP, 128), jnp.int32),
            pltpu.VMEM((2, _GRP, 128), jnp.int32),
            pltpu.VMEM((2, _GRP, 128, 32), _f32),
            pltpu.VMEM_SHARED((_NPAD, 32), _f32),
            pltpu.SemaphoreType.DMA,
            pltpu.SemaphoreType.DMA,
            pltpu.SemaphoreType.DMA,
            pltpu.SemaphoreType.DMA,
        ],
        compiler_params=pltpu.CompilerParams(use_tc_tiling_on_sc=False),
    )
    def k(tab_hbm, src_hbm, dst_hbm, z_hbm, out_hbm,
          sidx, didx, rows_v, acc, gsem, ssem, isem_s, isem_d):
        c = lax.axis_index("c")
        s = lax.axis_index("s")
        task = (s * 2 + c) if by_wid else s
        nrows = _NPAD // 16
        pltpu.sync_copy(z_hbm.at[pl.ds(s * nrows, nrows)],
                        acc.at[pl.ds(s * nrows, nrows)])
        plsc.subcore_barrier()
        base = task * rows_per_task
        ngrp = rows_per_task // _GRP
        pltpu.sync_copy(src_hbm.at[c, pl.ds(base, _GRP)], sidx.at[0])
        pltpu.sync_copy(dst_hbm.at[pl.ds(base, _GRP)], didx.at[0])

        def body(g, carry):
            p = g % 2
            pn = (g + 1) % 2

            @pl.when(g >= 1)
            def _():
                pltpu.make_async_copy(
                    src_hbm.at[c, pl.ds(base, _GRP)], sidx.at[p], isem_s).wait()
                pltpu.make_async_copy(
                    dst_hbm.at[pl.ds(base, _GRP)], didx.at[p], isem_d).wait()
            for u in range(_GRP):
                pltpu.async_copy(
                    tab_hbm.at[sidx.at[p, u]], rows_v.at[p, u], gsem)
            for u in range(_GRP):
                pltpu.make_async_copy(
                    tab_hbm.at[sidx.at[p, u]], rows_v.at[p, u], gsem).wait()

            @pl.when(g >= 1)
            def _():
                for u in range(_GRP):
                    pltpu.make_async_copy(
                        z_hbm.at[pl.ds(0, 128)], rows_v.at[pn, u], ssem).wait()

            @pl.when(g + 1 < ngrp)
            def _():
                nb = base + (g + 1) * _GRP
                pltpu.async_copy(src_hbm.at[c, pl.ds(nb, _GRP)],
                                 sidx.at[pn], isem_s)
                pltpu.async_copy(dst_hbm.at[pl.ds(nb, _GRP)],
                                 didx.at[pn], isem_d)
            for u in range(_GRP):
                pltpu.async_copy(rows_v.at[p, u], acc.at[didx.at[p, u]],
                                 ssem, add=True)
            return carry

        lax.fori_loop(0, ngrp, body, 0)
        pl_last = (ngrp - 1) % 2
        for u in range(_GRP):
            pltpu.make_async_copy(
                z_hbm.at[pl.ds(0, 128)], rows_v.at[pl_last, u], ssem).wait()
        plsc.subcore_barrier()
        pltpu.sync_copy(acc.at[pl.ds(s * nrows, nrows)],
                        out_hbm.at[c, pl.ds(s * nrows, nrows)])

    return k


def _tc_prescale(xflat, degt):
    def body(x_ref, d_ref, xa_ref, xb_ref):
        d = d_ref[:, 0:1] + d_ref[:, 1:2] + 1.0
        dinv = lax.rsqrt(d)
        x = x_ref[...]
        xa_ref[...] = x[:, 0:32] * dinv
        xb_ref[...] = x[:, 32:64] * dinv

    return pl.pallas_call(
        body,
        grid=(_NPAD // _R,),
        in_specs=[
            pl.BlockSpec((_R, 64), lambda i: (i, 0)),
            pl.BlockSpec((_R, 2), lambda i: (i, 0)),
        ],
        out_specs=[pl.BlockSpec((_R, 32), lambda i: (i, 0))] * 2,
        out_shape=[jax.ShapeDtypeStruct((_NPAD, 32), _f32)] * 2,
    )(xflat, degt)


def _softmax4(att):
    a = att
    e = jnp.exp(a - jnp.max(a))
    return e / jnp.sum(e)


def _mmh(w, x):
    return lax.dot_general(w, x, (((1,), (0,)), ((), ())),
                           preferred_element_type=_f32,
                           precision=lax.Precision.HIGHEST)


def _mmb(w, x):
    return lax.dot_general(w.astype(jnp.bfloat16), x.astype(jnp.bfloat16),
                           (((1,), (0,)), ((), ())),
                           preferred_element_type=_f32)


def _gru_step_t(g, Hs, lwz, lwr, lwh, lbz, lbr, lbh):
    gz, gr, gh = g[0:32], g[32:64], g[64:96]
    z = jax.nn.sigmoid(_mmb(lwz, jnp.concatenate([gz, Hs], axis=0)) + lbz)
    r = jax.nn.sigmoid(_mmb(lwr, jnp.concatenate([gr, Hs], axis=0)) + lbr)
    ht = jnp.tanh(_mmb(lwh, jnp.concatenate([gh, Hs * r], axis=0)) + lbh)
    return z * Hs + (1.0 - z) * ht


def _tc_layer1(y1pT, xT, degq, wcT, bcT, lwzT, lwrT, lwhT, lbzT, lbrT, lbhT,
               att):
    def body(y_ref, x_ref, d_ref, wc_ref, bc_ref, lwz_ref, lwr_ref,
             lwh_ref, lbz_ref, lbr_ref, lbh_ref, att_ref, hq_ref, yt_ref):
        d = d_ref[0:1, :] + d_ref[1:2, :] + 1.0
        dinv = lax.rsqrt(d)
        yadj = jnp.concatenate([y_ref[0], y_ref[1]], axis=0)
        Y1 = yadj * dinv + x_ref[...] * (dinv * dinv)
        p = _softmax4(att_ref[...])
        wc = wc_ref[...]
        bc = bc_ref[...]
        lwz_, lwr_, lwh_ = lwz_ref[...], lwr_ref[...], lwh_ref[...]
        lbz_, lbr_, lbh_ = lbz_ref[...], lbr_ref[...], lbh_ref[...]
        Hs = jnp.zeros((32, _R), _f32)
        acc = jnp.zeros((32, _R), _f32)
        for t in range(_T):
            g = _mmh(wc, Y1[t * 16:(t + 1) * 16]) + bc
            Hs = _gru_step_t(g, Hs, lwz_, lwr_, lwh_, lbz_, lbr_, lbh_)
            acc = acc + p[0:1, t:t + 1] * Hs
        h = jnp.maximum(acc, 0.0).astype(jnp.bfloat16).astype(_f32)
        hq_ref[...] = h * dinv
        yt_ref[...] = jnp.concatenate(
            [Y1[15:16], Y1[31:32], Y1[47:48], Y1[63:64]], axis=0)

    full = lambda shape: pl.BlockSpec(shape, lambda i: tuple(0 for _ in shape))
    return pl.pallas_call(
        body,
        grid=(_NPAD // _R,),
        in_specs=[
            pl.BlockSpec((2, 32, _R), lambda i: (0, 0, i)),
            pl.BlockSpec((64, _R), lambda i: (0, i)),
            pl.BlockSpec((2, _R), lambda i: (0, i)),
            full((96, 16)), full((96, 1)),
            full((32, 64)), full((32, 64)), full((32, 64)),
            full((32, 1)), full((32, 1)), full((32, 1)),
            full((1, 4)),
        ],
        out_specs=[
            pl.BlockSpec((32, _R), lambda i: (0, i)),
            pl.BlockSpec((4, _R), lambda i: (0, i)),
        ],
        out_shape=[
            jax.ShapeDtypeStruct((32, _NPAD), _f32),
            jax.ShapeDtypeStruct((4, _NPAD), _f32),
        ],
    )(y1pT, xT, degq, wcT, bcT, lwzT, lwrT, lwhT, lbzT, lbrT, lbhT, att)


def _tc_layer2(y2pT, hqT, ytime, degq, w2topT, w2lastT, bc2T, lwzT, lwrT,
               lwhT, lbzT, lbrT, lbhT, att2, w1T, b1T, w2T, b2T):
    def body(y_ref, hq_ref, yt_ref, d_ref, wt_ref, wl_ref, bc_ref,
             lwz_ref, lwr_ref, lwh_ref, lbz_ref, lbr_ref, lbh_ref, att_ref,
             w1_ref, b1_ref, w2_ref, b2_ref, out_ref):
        d = d_ref[0:1, :] + d_ref[1:2, :] + 1.0
        dinv = lax.rsqrt(d)
        yh = (y_ref[0] + y_ref[1] + hq_ref[...]) * dinv
        p = _softmax4(att_ref[...])
        gbase = _mmh(wt_ref[...], yh)
        wl = wl_ref[...]
        bc = bc_ref[...]
        lwz_, lwr_, lwh_ = lwz_ref[...], lwr_ref[...], lwh_ref[...]
        lbz_, lbr_, lbh_ = lbz_ref[...], lbr_ref[...], lbh_ref[...]
        yt = yt_ref[...]
        Hs = jnp.zeros((32, _R), _f32)
        acc = jnp.zeros((32, _R), _f32)
        for t in range(_T):
            g = gbase + wl * yt[t:t + 1, :] + bc
            Hs = _gru_step_t(g, Hs, lwz_, lwr_, lwh_, lbz_, lbr_, lbh_)
            acc = acc + p[0:1, t:t + 1] * Hs
        h = jnp.maximum(acc, 0.0)
        m = jnp.maximum(_mmb(w1_ref[...], h) + b1_ref[...], 0.0)
        out_ref[...] = _mmb(w2_ref[...], m) + b2_ref[...]

    full = lambda shape: pl.BlockSpec(shape, lambda i: tuple(0 for _ in shape))
    return pl.pallas_call(
        body,
        grid=(_NPAD // _R,),
        in_specs=[
            pl.BlockSpec((2, 32, _R), lambda i: (0, 0, i)),
            pl.BlockSpec((32, _R), lambda i: (0, i)),
            pl.BlockSpec((4, _R), lambda i: (0, i)),
            pl.BlockSpec((2, _R), lambda i: (0, i)),
            full((96, 32)), full((96, 1)), full((96, 1)),
            full((32, 64)), full((32, 64)), full((32, 64)),
            full((32, 1)), full((32, 1)), full((32, 1)),
            full((1, 4)),
            full((32, 32)), full((32, 1)), full((8, 32)), full((8, 1)),
        ],
        out_specs=pl.BlockSpec((8, _R), lambda i: (0, i)),
        out_shape=jax.ShapeDtypeStruct((8, _NPAD), _f32),
    )(y2pT, hqT, ytime, degq, w2topT, w2lastT, bc2T, lwzT, lwrT, lwhT,
      lbzT, lbrT, lbhT, att2, w1T, b1T, w2T, b2T)


@jax.jit
def _run(x_1, edge_index_1, params1, params2, lin):
    src = edge_index_1[0]
    dst = edge_index_1[1]
    padi = jnp.full((_EPAD - _E,), _N, jnp.int32)
    src_r = jnp.concatenate([src.astype(jnp.int32), padi]).reshape(_RIDX, 128)
    dst_r = jnp.concatenate([dst.astype(jnp.int32), padi]).reshape(_RIDX, 128)

    xflat = x_1.transpose(0, 2, 1).reshape(_N, _T * _IN)
    xflat = xflat.astype(jnp.bfloat16).astype(_f32)
    xflat = jnp.pad(xflat, ((0, _NPAD - _N), (0, 0)))

    z1 = jnp.zeros((_NPAD,), _f32)
    z32 = jnp.zeros((_NPAD, 32), _f32)

    p1, p2 = params1, params2
    def q16(w):
        return w.astype(jnp.bfloat16).astype(_f32)

    wcT1 = q16(jnp.concatenate([p1["Wz"], p1["Wr"], p1["Wh"]], axis=1).T)
    bcT1 = jnp.concatenate([p1["bz"], p1["br"], p1["bh"]])[:, None]
    att1 = p1["att"][None, :]
    wcat2 = jnp.concatenate([p2["Wz"], p2["Wr"], p2["Wh"]], axis=1)
    w2topT = q16(wcat2[0:32].T)
    w2lastT = q16(wcat2[32:33].T)
    bc2T = jnp.concatenate([p2["bz"], p2["br"], p2["bh"]])[:, None]
    att2 = p2["att"][None, :]
    w2T = jnp.pad(lin["W2"], ((0, 0), (0, 4))).T
    b2T = jnp.pad(lin["b2"], (0, 4))[:, None]

    def lwT(p):
        return (p["LWz"].T, p["LWr"].T, p["LWh"].T,
                p["Lbz"][:, None], p["Lbr"][:, None], p["Lbh"][:, None])

    degp = _sc_degree(dst_r, z1)
    degq = degp.reshape(2, _NPAD)
    degt = degq.T
    xa, xb = _tc_prescale(xflat, degt)
    xab = jnp.concatenate([xa, xb], axis=0)
    src3_p1 = jnp.stack([src_r, src_r + _NPAD])
    src3_p2 = jnp.stack([src_r, src_r])
    y1p = _make_spmv(_ROWS_P1, False)(xab, src3_p1, dst_r, z32)
    xT = xflat.T
    hqT, ytime = _tc_layer1(y1p.transpose(0, 2, 1), xT, degq,
                            wcT1, bcT1, *lwT(p1), att1)
    hq = hqT.T
    y2p = _make_spmv(_ROWS_P2, True)(hq, src3_p2, dst_r, z32)
    predT = _tc_layer2(y2p.transpose(0, 2, 1), hqT, ytime, degq,
                       w2topT, w2lastT, bc2T, *lwT(p2), att2,
                       lin["W1"].T, lin["b1"][:, None], w2T, b2T)
    return predT.T[:_N, 0:4]


def kernel(x_1, edge_index_1, x_2, edge_index_2, params1, params2, lin):
    return _run(x_1, edge_index_1, params1, params2, lin)
